# baseline (speedup 1.0000x reference)
import jax
import jax.numpy as jnp
from jax.experimental import pallas as pl

_NUM_CATEGORIES = 256
_BATCH_BLOCK = 16
_LANE_BLOCK = 128


def _one_hot_block(idx_ref, out_ref):
    idx = idx_ref[...]
    cat = jax.lax.broadcasted_iota(
        jnp.int32, (idx.shape[0], _NUM_CATEGORIES, idx.shape[1]), 1)
    out_ref[...] = (idx[:, None, :] == cat).astype(jnp.float32)


def kernel(indices):
    batch, seq = indices.shape
    bblk = _BATCH_BLOCK
    lblk = _LANE_BLOCK
    grid = (batch // bblk, pl.cdiv(seq, lblk))
    return pl.pallas_call(
        _one_hot_block,
        grid=grid,
        in_specs=[pl.BlockSpec((bblk, lblk), lambda i, j: (i, j))],
        out_specs=pl.BlockSpec((bblk, _NUM_CATEGORIES, lblk),
                               lambda i, j: (i, 0, j)),
        out_shape=jax.ShapeDtypeStruct((batch, _NUM_CATEGORIES, seq), jnp.float32),
    )(indices)


# speedup vs baseline: 1.0994x; 1.0994x over previous
import jax
import jax.numpy as jnp
from jax.experimental import pallas as pl

_NUM_CATEGORIES = 256
_BATCH_BLOCK = 16


def _one_hot_block(idx_ref, out_ref):
    idx = idx_ref[...]
    cat = jax.lax.broadcasted_iota(
        jnp.int32, (idx.shape[0], _NUM_CATEGORIES, idx.shape[1]), 1)
    out_ref[...] = (idx[:, None, :] == cat).astype(jnp.float32)


def kernel(indices):
    batch, seq = indices.shape
    bblk = _BATCH_BLOCK
    grid = (batch // bblk,)
    return pl.pallas_call(
        _one_hot_block,
        grid=grid,
        in_specs=[pl.BlockSpec((bblk, seq), lambda i: (i, 0))],
        out_specs=pl.BlockSpec((bblk, _NUM_CATEGORIES, seq), lambda i: (i, 0, 0)),
        out_shape=jax.ShapeDtypeStruct((batch, _NUM_CATEGORIES, seq), jnp.float32),
    )(indices)
